# baseline (speedup 1.0000x reference)
import functools
import math

import jax
import jax.numpy as jnp
from jax import lax
from jax.experimental import pallas as pl
from jax.experimental.pallas import tpu as pltpu
from jax.experimental.pallas import tpu_sc as plsc

CUTOFF = 5.0
RBF = 32
D = 128
NA = 100
ALPHA = 5.0 / CUTOFF
_START = math.exp(-CUTOFF)
_DELTA = (1.0 - _START) / (RBF - 1)
GAMMA = (2.0 * _DELTA) ** (-2)
S3 = math.sqrt(3.0)

BR = 32
BE = BR * 128
BEC = 4000
BN = 2000

NC = 2
NS = 16
NW = NC * NS
KE = 80


def _f32(x):
    return x.astype(jnp.float32)


def _cutoff(r):
    return ((jnp.cos((math.pi / CUTOFF) * r) + 1.0) * 0.5) * _f32(r < CUTOFF)


def _rbf(r, centers):
    xe = jnp.exp(-ALPHA * r)
    d = xe - centers
    return jnp.exp(-GAMMA * (d * d)) * _cutoff(r)


KC = 40
KA = 2 * KC + 16
NW3 = 3 * D


def _edge_dense_body(px_ref, py_ref, pz_ref, cen_ref, wc_ref,
                     r0_ref, r1_ref, r2_ref, endp_ref, eerp_ref):
    x = px_ref[...]
    y = py_ref[...]
    z3 = pz_ref[...]
    r = jnp.sqrt(x * x + y * y + z3 * z3)
    inv = 1.0 / r
    ux = x * inv
    uy = y * inv
    uz = z3 * inv
    s4 = S3 * ux * uz
    s5 = S3 * ux * uy
    s6 = uy * uy - 0.5 * (ux * ux + uz * uz)
    s7 = S3 * uy * uz
    s8 = (S3 / 2.0) * (uz * uz - ux * ux)
    cut = _cutoff(r)
    xe = jnp.exp(-ALPHA * r)
    cenT = cen_ref[...]
    wc = wc_ref[...]
    zero7 = jnp.zeros((KC - RBF - 1, 128), jnp.float32)
    one1 = jnp.ones((1, 128), jnp.float32)
    for i in range(BR):
        sl = slice(i, i + 1)
        xe_i = xe[sl, :]
        cut_i = cut[sl, :]
        dd = cenT - xe_i
        g = jnp.exp(-GAMMA * (dd * dd))
        gn = g * (cut_i * cut_i)
        ge = g * cut_i
        blk = jnp.concatenate(
            [gn, cut_i, zero7, ge, one1, zero7,
             r[sl, :], uy[sl, :], uz[sl, :], ux[sl, :],
             s4[sl, :], s5[sl, :], s6[sl, :], s7[sl, :], s8[sl, :],
             zero7], axis=0)
        out = lax.dot_general(blk, wc, (((0,), (0,)), ((), ())),
                              preferred_element_type=jnp.float32)
        rb = pl.ds(128 * i, 128)
        endp_ref[rb, :] = out[:, :D]
        eerp_ref[rb, :] = out[:, D:2 * D]
        r0_ref[rb, :] = out[:, 2 * D:2 * D + 1]
        r1_ref[rb, :] = out[:, 2 * D + 1:2 * D + 4]
        r2_ref[rb, :] = out[:, 2 * D + 4:2 * D + 9]


def _edge_dense(px, py, pz, centers_col, wcomb, e):
    rows = px.shape[0]
    epad = rows * 128
    g = rows // BR
    plane = pl.BlockSpec((BR, 128), lambda i: (i, 0))
    return pl.pallas_call(
        _edge_dense_body,
        grid=(g,),
        in_specs=[
            plane, plane, plane,
            pl.BlockSpec((RBF, 1), lambda i: (0, 0)),
            pl.BlockSpec((KA, NW3), lambda i: (0, 0)),
        ],
        out_specs=[
            pl.BlockSpec((BE, 1), lambda i: (i, 0)),
            pl.BlockSpec((BE, 3), lambda i: (i, 0)),
            pl.BlockSpec((BE, 5), lambda i: (i, 0)),
            pl.BlockSpec((BE, D), lambda i: (i, 0)),
            pl.BlockSpec((BE, D), lambda i: (i, 0)),
        ],
        out_shape=[
            jax.ShapeDtypeStruct((e, 1), jnp.float32),
            jax.ShapeDtypeStruct((e, 3), jnp.float32),
            jax.ShapeDtypeStruct((e, 5), jnp.float32),
            jax.ShapeDtypeStruct((epad, D), jnp.float32),
            jax.ShapeDtypeStruct((epad, D), jnp.float32),
        ],
    )(px, py, pz, centers_col, wcomb)


def _node_embed_body(z_ref, anbr_ref, anode_ref):
    zb = z_ref[...]
    ids = lax.broadcasted_iota(jnp.int32, (1, NA), 1)
    oh = _f32(zb == ids)
    anode_ref[...] = jnp.dot(oh, anbr_ref[...],
                             preferred_element_type=jnp.float32)


def _node_embed(z2, a_nbr):
    n = z2.shape[0]
    g = n // BN
    return pl.pallas_call(
        _node_embed_body,
        grid=(g,),
        in_specs=[
            pl.BlockSpec((BN, 1), lambda i: (i, 0)),
            pl.BlockSpec((NA, D), lambda i: (0, 0)),
        ],
        out_specs=pl.BlockSpec((BN, D), lambda i: (i, 0)),
        out_shape=jax.ShapeDtypeStruct((n, D), jnp.float32),
    )(z2, a_nbr)


def _sc_scatter_body(anode, nj, ni, endp, out,
                     accum, idxj0, idxi0, rows0, erow0, semg0, seme0,
                     idxj1, idxi1, rows1, erow1, semg1, seme1, zbuf):
    c = lax.axis_index("c")
    s = lax.axis_index("s")
    wid = s * NC + c
    n_nodes = accum.shape[0]
    epw = nj.shape[0] // NW
    nblk = epw // KE
    chunk = 1000
    ncp = n_nodes // chunk
    zrows = zbuf.shape[0]

    def _zfill(i, _):
        for j in range(D // 16):
            zbuf[i, pl.ds(16 * j, 16)] = jnp.zeros((16,), jnp.float32)
        return 0
    lax.fori_loop(0, zrows, _zfill, 0)

    @pl.when(s < ncp)
    def _():
        def _zcopy(k, _):
            pltpu.sync_copy(zbuf, accum.at[pl.ds(s * chunk + k * zrows, zrows)])
            return 0
        lax.fori_loop(0, chunk // zrows, _zcopy, 0)

    plsc.subcore_barrier()

    base = wid * epw

    def _start(off, idxj, idxi, rows, erow, semg, seme):
        pltpu.sync_copy(nj.at[pl.ds(off, KE)], idxj)
        pltpu.sync_copy(ni.at[pl.ds(off, KE)], idxi)
        pltpu.async_copy(anode.at[idxj], rows, semg)
        pltpu.async_copy(endp.at[pl.ds(off, KE)], erow, seme)

    def _finish(idxj, idxi, rows, erow, semg, seme):
        pltpu.make_async_copy(anode.at[idxj], rows, semg).wait()
        pltpu.make_async_copy(endp.at[pl.ds(0, KE)], erow, seme).wait()

        def _mulrow(r, _):
            for j in range(D // 16):
                sl = pl.ds(16 * j, 16)
                rows[r, sl] = rows[r, sl] * erow[r, sl]
            return 0
        lax.fori_loop(0, KE, _mulrow, 0)
        pltpu.sync_copy(rows, accum.at[idxi], add=True)

    def _blk(b, _):
        _start(base + b * KE, idxj0, idxi0, rows0, erow0, semg0, seme0)
        _finish(idxj0, idxi0, rows0, erow0, semg0, seme0)
        return 0
    lax.fori_loop(0, nblk, _blk, 0)

    plsc.subcore_barrier()

    @pl.when(s < ncp)
    def _():
        pltpu.sync_copy(accum.at[pl.ds(s * chunk, chunk)],
                        out.at[c, pl.ds(s * chunk, chunk)])


def _sc_scatter_partials(anode, nj, ni, endp):
    n = anode.shape[0]
    bufset = [
        pltpu.VMEM((KE,), jnp.int32),
        pltpu.VMEM((KE,), jnp.int32),
        pltpu.VMEM((KE, D), jnp.float32),
        pltpu.VMEM((KE, D), jnp.float32),
        pltpu.SemaphoreType.DMA,
        pltpu.SemaphoreType.DMA,
    ]
    f = pl.kernel(
        _sc_scatter_body,
        out_type=jax.ShapeDtypeStruct((NC, n, D), jnp.float32),
        mesh=plsc.VectorSubcoreMesh(core_axis_name="c", subcore_axis_name="s",
                                    num_cores=NC, num_subcores=NS),
        scratch_types=[pltpu.VMEM_SHARED((n, D), jnp.float32)]
        + bufset + bufset + [pltpu.VMEM((125, D), jnp.float32)],
    )
    return f(anode, nj, ni, endp)


def _node_mlp_body(z_ref, m0_ref, m1_ref, ana_ref, w1_ref, b1_ref,
                   g_ref, bb_ref, w2_ref, b2_ref, h_ref):
    zb = z_ref[...]
    ids = lax.broadcasted_iota(jnp.int32, (1, NA), 1)
    oh = _f32(zb == ids)
    a = jnp.dot(oh, ana_ref[...], preferred_element_type=jnp.float32)
    m = m0_ref[...] + m1_ref[...]
    pre = (jnp.dot(a, w1_ref[0:D, :], preferred_element_type=jnp.float32)
           + jnp.dot(m, w1_ref[D:2 * D, :], preferred_element_type=jnp.float32)
           + b1_ref[...])
    mu = jnp.mean(pre, axis=1, keepdims=True)
    cen = pre - mu
    var = jnp.mean(cen * cen, axis=1, keepdims=True)
    xn = cen / jnp.sqrt(var + 1e-5) * g_ref[...] + bb_ref[...]
    h1 = xn / (1.0 + jnp.exp(-xn))
    h_ref[...] = (jnp.dot(h1, w2_ref[...], preferred_element_type=jnp.float32)
                  + b2_ref[...])


def _node_mlp(z2, m0, m1, a_na, w1, b1, ln_g, ln_b, w2, b2):
    n = z2.shape[0]
    g = n // BN
    full = lambda shape: pl.BlockSpec(shape, lambda i: tuple(0 for _ in shape))
    return pl.pallas_call(
        _node_mlp_body,
        grid=(g,),
        in_specs=[
            pl.BlockSpec((BN, 1), lambda i: (i, 0)),
            pl.BlockSpec((BN, D), lambda i: (i, 0)),
            pl.BlockSpec((BN, D), lambda i: (i, 0)),
            full((NA, D)),
            full((2 * D, D)),
            full((1, D)),
            full((1, D)),
            full((1, D)),
            full((D, D)),
            full((1, D)),
        ],
        out_specs=pl.BlockSpec((BN, D), lambda i: (i, 0)),
        out_shape=jax.ShapeDtypeStruct((n, D), jnp.float32),
    )(z2, m0, m1, a_na, w1, b1, ln_g, ln_b, w2, b2)


def _sc_gather_body(h, ni, nj, out,
                    idxi0, idxj0, rowsa0, rowsb0, sema0, semb0,
                    idxi1, idxj1, rowsa1, rowsb1, sema1, semb1):
    c = lax.axis_index("c")
    s = lax.axis_index("s")
    wid = s * NC + c
    epw = ni.shape[0] // NW
    nblk = epw // KE
    base = wid * epw

    def _start(off, idxi, idxj, rowsa, rowsb, sema, semb):
        pltpu.sync_copy(ni.at[pl.ds(off, KE)], idxi)
        pltpu.sync_copy(nj.at[pl.ds(off, KE)], idxj)
        pltpu.async_copy(h.at[idxi], rowsa, sema)
        pltpu.async_copy(h.at[idxj], rowsb, semb)

    def _finish(off, idxi, idxj, rowsa, rowsb, sema, semb):
        pltpu.make_async_copy(h.at[idxi], rowsa, sema).wait()
        pltpu.make_async_copy(h.at[idxj], rowsb, semb).wait()

        def _addrow(r, _):
            for j in range(D // 16):
                sl = pl.ds(16 * j, 16)
                rowsa[r, sl] = rowsa[r, sl] + rowsb[r, sl]
            return 0
        lax.fori_loop(0, KE, _addrow, 0)
        pltpu.sync_copy(rowsa, out.at[pl.ds(off, KE)])

    _start(base, idxi0, idxj0, rowsa0, rowsb0, sema0, semb0)

    def _pair(k, _):
        b0 = base + 2 * k * KE
        _start(b0 + KE, idxi1, idxj1, rowsa1, rowsb1, sema1, semb1)
        _finish(b0, idxi0, idxj0, rowsa0, rowsb0, sema0, semb0)
        _start(b0 + 2 * KE, idxi0, idxj0, rowsa0, rowsb0, sema0, semb0)
        _finish(b0 + KE, idxi1, idxj1, rowsa1, rowsb1, sema1, semb1)
        return 0
    lax.fori_loop(0, (nblk - 1) // 2, _pair, 0)
    _finish(base + (nblk - 1) * KE, idxi0, idxj0, rowsa0, rowsb0,
            sema0, semb0)


def _sc_gather_pairsum(h, ni, nj):
    e = ni.shape[0]
    bufset = [
        pltpu.VMEM((KE,), jnp.int32),
        pltpu.VMEM((KE,), jnp.int32),
        pltpu.VMEM((KE, D), jnp.float32),
        pltpu.VMEM((KE, D), jnp.float32),
        pltpu.SemaphoreType.DMA,
        pltpu.SemaphoreType.DMA,
    ]
    f = pl.kernel(
        _sc_gather_body,
        out_type=jax.ShapeDtypeStruct((e, D), jnp.float32),
        mesh=plsc.VectorSubcoreMesh(core_axis_name="c", subcore_axis_name="s",
                                    num_cores=NC, num_subcores=NS),
        scratch_types=bufset + bufset,
    )
    return f(h, ni, nj)


def _edge_out_body(hs_ref, eerp_ref, t_ref):
    t_ref[...] = hs_ref[...] * eerp_ref[...]


def _edge_out(hs, eerp):
    e = hs.shape[0]
    g = e // BEC
    blk = pl.BlockSpec((BEC, D), lambda i: (i, 0))
    return pl.pallas_call(
        _edge_out_body,
        grid=(g,),
        in_specs=[blk, blk],
        out_specs=blk,
        out_shape=jax.ShapeDtypeStruct((e, D), jnp.float32),
    )(hs, eerp)


def kernel(z, p, edge_index, W_ndp, b_ndp, A_nbr, A_na, W1, b1, ln_g, ln_b,
           W2, b2, W_erp, b_erp):
    centers_col = jnp.linspace(_START, 1.0, RBF,
                               dtype=jnp.float32).reshape(RBF, 1)
    zrow = jnp.zeros((KC - RBF - 1, D), jnp.float32)
    w_top = jnp.concatenate([W_ndp, b_ndp.reshape(1, D), zrow], axis=0)
    w_bot = jnp.concatenate([W_erp, b_erp.reshape(1, D), zrow], axis=0)
    zblk = jnp.zeros((KC, D), jnp.float32)
    sel = jnp.concatenate(
        [jnp.zeros((2 * KC, D), jnp.float32),
         jnp.eye(16, D, dtype=jnp.float32)], axis=0)
    z16 = jnp.zeros((16, D), jnp.float32)
    wcomb = jnp.concatenate(
        [jnp.concatenate([w_top, zblk, z16], axis=0),
         jnp.concatenate([zblk, w_bot, z16], axis=0),
         sel], axis=1)

    nj = edge_index[0]
    ni = edge_index[1]
    z2 = z.reshape(-1, 1)
    e = p.shape[0]
    rows = e // 128
    rpad = ((rows + BR - 1) // BR) * BR
    extra = rpad * 128 - e
    px = jnp.pad(p[:, 0], (0, extra)).reshape(rpad, 128)
    py = jnp.pad(p[:, 1], (0, extra)).reshape(rpad, 128)
    pz = jnp.pad(p[:, 2], (0, extra), constant_values=1.0).reshape(rpad, 128)

    r_0, r_ij_1, r_ij_2, e_ndp, e_erp = _edge_dense(
        px, py, pz, centers_col, wcomb, e)

    anode = _node_embed(z2, A_nbr)
    m_part = _sc_scatter_partials(anode, nj, ni, e_ndp)
    h = _node_mlp(z2, m_part[0], m_part[1], A_na, W1, b1.reshape(1, D),
                  ln_g.reshape(1, D), ln_b.reshape(1, D), W2, b2.reshape(1, D))
    hs = _sc_gather_pairsum(h, ni, nj)
    t_ij = _edge_out(hs, e_erp)
    return (r_0, r_ij_1, r_ij_2, h, t_ij)


# speedup vs baseline: 1.1975x; 1.1975x over previous
import functools
import math

import jax
import jax.numpy as jnp
from jax import lax
from jax.experimental import pallas as pl
from jax.experimental.pallas import tpu as pltpu
from jax.experimental.pallas import tpu_sc as plsc

CUTOFF = 5.0
RBF = 32
D = 128
NA = 100
ALPHA = 5.0 / CUTOFF
_START = math.exp(-CUTOFF)
_DELTA = (1.0 - _START) / (RBF - 1)
GAMMA = (2.0 * _DELTA) ** (-2)
S3 = math.sqrt(3.0)

BR = 32
BE = BR * 128
BEC = 4000
BN = 2000

NC = 2
NS = 16
NW = NC * NS
KE = 80


def _f32(x):
    return x.astype(jnp.float32)


def _cutoff(r):
    return ((jnp.cos((math.pi / CUTOFF) * r) + 1.0) * 0.5) * _f32(r < CUTOFF)


def _rbf(r, centers):
    xe = jnp.exp(-ALPHA * r)
    d = xe - centers
    return jnp.exp(-GAMMA * (d * d)) * _cutoff(r)


KC = 40
KA = 2 * KC + 16
NW3 = 3 * D


def _edge_dense_body(px_ref, py_ref, pz_ref, cen_ref, wc_ref,
                     sph_ref, endp_ref, eerp_ref):
    x = px_ref[...]
    y = py_ref[...]
    z3 = pz_ref[...]
    r = jnp.sqrt(x * x + y * y + z3 * z3)
    inv = 1.0 / r
    ux = x * inv
    uy = y * inv
    uz = z3 * inv
    sph_ref[0] = r
    sph_ref[1] = uy
    sph_ref[2] = uz
    sph_ref[3] = ux
    sph_ref[4] = S3 * ux * uz
    sph_ref[5] = S3 * ux * uy
    sph_ref[6] = uy * uy - 0.5 * (ux * ux + uz * uz)
    sph_ref[7] = S3 * uy * uz
    sph_ref[8] = (S3 / 2.0) * (uz * uz - ux * ux)
    cut = _cutoff(r)
    xe = jnp.exp(-ALPHA * r)
    cenT = cen_ref[...]
    wc = wc_ref[...]
    zero7 = jnp.zeros((KC - RBF - 1, 128), jnp.float32)
    one1 = jnp.ones((1, 128), jnp.float32)
    for i in range(BR):
        sl = slice(i, i + 1)
        xe_i = xe[sl, :]
        cut_i = cut[sl, :]
        dd = cenT - xe_i
        g = jnp.exp(-GAMMA * (dd * dd))
        gn = g * (cut_i * cut_i)
        ge = g * cut_i
        blk = jnp.concatenate(
            [gn, cut_i, zero7, ge, one1, zero7], axis=0)
        out = lax.dot_general(blk, wc, (((0,), (0,)), ((), ())),
                              preferred_element_type=jnp.float32)
        rb = pl.ds(128 * i, 128)
        endp_ref[rb, :] = out[:, :D]
        eerp_ref[rb, :] = out[:, D:]


def _edge_dense(px, py, pz, centers_col, wcomb):
    rows = px.shape[0]
    e = rows * 128
    g = rows // BR
    plane = pl.BlockSpec((BR, 128), lambda i: (i, 0))
    return pl.pallas_call(
        _edge_dense_body,
        grid=(g,),
        in_specs=[
            plane, plane, plane,
            pl.BlockSpec((RBF, 1), lambda i: (0, 0)),
            pl.BlockSpec((2 * KC, 2 * D), lambda i: (0, 0)),
        ],
        out_specs=[
            pl.BlockSpec((9, BR, 128), lambda i: (0, i, 0)),
            pl.BlockSpec((BE, D), lambda i: (i, 0)),
            pl.BlockSpec((BE, D), lambda i: (i, 0)),
        ],
        out_shape=[
            jax.ShapeDtypeStruct((9, rows, 128), jnp.float32),
            jax.ShapeDtypeStruct((e, D), jnp.float32),
            jax.ShapeDtypeStruct((e, D), jnp.float32),
        ],
    )(px, py, pz, centers_col, wcomb)


def _node_embed_body(z_ref, anbr_ref, anode_ref):
    zb = z_ref[...]
    ids = lax.broadcasted_iota(jnp.int32, (1, NA), 1)
    oh = _f32(zb == ids)
    anode_ref[...] = jnp.dot(oh, anbr_ref[...],
                             preferred_element_type=jnp.float32)


def _node_embed(z2, a_nbr):
    n = z2.shape[0]
    g = n // BN
    return pl.pallas_call(
        _node_embed_body,
        grid=(g,),
        in_specs=[
            pl.BlockSpec((BN, 1), lambda i: (i, 0)),
            pl.BlockSpec((NA, D), lambda i: (0, 0)),
        ],
        out_specs=pl.BlockSpec((BN, D), lambda i: (i, 0)),
        out_shape=jax.ShapeDtypeStruct((n, D), jnp.float32),
    )(z2, a_nbr)


def _sc_scatter_body(anode, nj, ni, endp, out,
                     accum, idxj0, idxi0, rows0, erow0, semg0,
                     idxj1, idxi1, rows1, erow1, semg1, zbuf):
    c = lax.axis_index("c")
    s = lax.axis_index("s")
    wid = s * NC + c
    n_nodes = accum.shape[0]
    epw = nj.shape[0] // NW
    nblk = epw // KE
    chunk = 1000
    ncp = n_nodes // chunk
    zrows = zbuf.shape[0]

    def _zfill(i, _):
        for j in range(D // 16):
            zbuf[i, pl.ds(16 * j, 16)] = jnp.zeros((16,), jnp.float32)
        return 0
    lax.fori_loop(0, zrows, _zfill, 0)

    @pl.when(s < ncp)
    def _():
        def _zcopy(k, _):
            pltpu.sync_copy(zbuf, accum.at[pl.ds(s * chunk + k * zrows, zrows)])
            return 0
        lax.fori_loop(0, chunk // zrows, _zcopy, 0)

    plsc.subcore_barrier()

    base = wid * epw

    def _start(off, idxj, idxi, rows, semg):
        pltpu.sync_copy(nj.at[pl.ds(off, KE)], idxj)
        pltpu.sync_copy(ni.at[pl.ds(off, KE)], idxi)
        pltpu.async_copy(anode.at[idxj], rows, semg)

    def _finish(off, idxj, idxi, rows, erow, semg):
        pltpu.sync_copy(endp.at[pl.ds(off, KE)], erow)
        pltpu.make_async_copy(anode.at[idxj], rows, semg).wait()

        def _mulrow(r, _):
            for j in range(D // 16):
                sl = pl.ds(16 * j, 16)
                rows[r, sl] = rows[r, sl] * erow[r, sl]
            return 0
        lax.fori_loop(0, KE, _mulrow, 0)
        pltpu.sync_copy(rows, accum.at[idxi], add=True)

    _start(base, idxj0, idxi0, rows0, semg0)

    def _pair(k, _):
        b0 = base + 2 * k * KE
        _start(b0 + KE, idxj1, idxi1, rows1, semg1)
        _finish(b0, idxj0, idxi0, rows0, erow0, semg0)
        _start(b0 + 2 * KE, idxj0, idxi0, rows0, semg0)
        _finish(b0 + KE, idxj1, idxi1, rows1, erow1, semg1)
        return 0
    lax.fori_loop(0, (nblk - 1) // 2, _pair, 0)
    _finish(base + (nblk - 1) * KE, idxj0, idxi0, rows0, erow0, semg0)

    plsc.subcore_barrier()

    @pl.when(s < ncp)
    def _():
        pltpu.sync_copy(accum.at[pl.ds(s * chunk, chunk)],
                        out.at[c, pl.ds(s * chunk, chunk)])


def _sc_scatter_partials(anode, nj, ni, endp):
    n = anode.shape[0]
    bufset = [
        pltpu.VMEM((KE,), jnp.int32),
        pltpu.VMEM((KE,), jnp.int32),
        pltpu.VMEM((KE, D), jnp.float32),
        pltpu.VMEM((KE, D), jnp.float32),
        pltpu.SemaphoreType.DMA,
    ]
    f = pl.kernel(
        _sc_scatter_body,
        out_type=jax.ShapeDtypeStruct((NC, n, D), jnp.float32),
        mesh=plsc.VectorSubcoreMesh(core_axis_name="c", subcore_axis_name="s",
                                    num_cores=NC, num_subcores=NS),
        scratch_types=[pltpu.VMEM_SHARED((n, D), jnp.float32)]
        + bufset + bufset + [pltpu.VMEM((40, D), jnp.float32)],
    )
    return f(anode, nj, ni, endp)


def _node_mlp_body(z_ref, m0_ref, m1_ref, ana_ref, w1_ref, b1_ref,
                   g_ref, bb_ref, w2_ref, b2_ref, h_ref):
    zb = z_ref[...]
    ids = lax.broadcasted_iota(jnp.int32, (1, NA), 1)
    oh = _f32(zb == ids)
    a = jnp.dot(oh, ana_ref[...], preferred_element_type=jnp.float32)
    m = m0_ref[...] + m1_ref[...]
    pre = (jnp.dot(a, w1_ref[0:D, :], preferred_element_type=jnp.float32)
           + jnp.dot(m, w1_ref[D:2 * D, :], preferred_element_type=jnp.float32)
           + b1_ref[...])
    mu = jnp.mean(pre, axis=1, keepdims=True)
    cen = pre - mu
    var = jnp.mean(cen * cen, axis=1, keepdims=True)
    xn = cen / jnp.sqrt(var + 1e-5) * g_ref[...] + bb_ref[...]
    h1 = xn / (1.0 + jnp.exp(-xn))
    h_ref[...] = (jnp.dot(h1, w2_ref[...], preferred_element_type=jnp.float32)
                  + b2_ref[...])


def _node_mlp(z2, m0, m1, a_na, w1, b1, ln_g, ln_b, w2, b2):
    n = z2.shape[0]
    g = n // BN
    full = lambda shape: pl.BlockSpec(shape, lambda i: tuple(0 for _ in shape))
    return pl.pallas_call(
        _node_mlp_body,
        grid=(g,),
        in_specs=[
            pl.BlockSpec((BN, 1), lambda i: (i, 0)),
            pl.BlockSpec((BN, D), lambda i: (i, 0)),
            pl.BlockSpec((BN, D), lambda i: (i, 0)),
            full((NA, D)),
            full((2 * D, D)),
            full((1, D)),
            full((1, D)),
            full((1, D)),
            full((D, D)),
            full((1, D)),
        ],
        out_specs=pl.BlockSpec((BN, D), lambda i: (i, 0)),
        out_shape=jax.ShapeDtypeStruct((n, D), jnp.float32),
    )(z2, m0, m1, a_na, w1, b1, ln_g, ln_b, w2, b2)


def _sc_gather_body(h, ni, nj, out,
                    idxi0, idxj0, rowsa0, rowsb0, sema0, semb0,
                    idxi1, idxj1, rowsa1, rowsb1, sema1, semb1):
    c = lax.axis_index("c")
    s = lax.axis_index("s")
    wid = s * NC + c
    epw = ni.shape[0] // NW
    nblk = epw // KE
    base = wid * epw

    def _start(off, idxi, idxj, rowsa, rowsb, sema, semb):
        pltpu.sync_copy(ni.at[pl.ds(off, KE)], idxi)
        pltpu.sync_copy(nj.at[pl.ds(off, KE)], idxj)
        pltpu.async_copy(h.at[idxi], rowsa, sema)
        pltpu.async_copy(h.at[idxj], rowsb, semb)

    def _finish(off, idxi, idxj, rowsa, rowsb, sema, semb):
        pltpu.make_async_copy(h.at[idxi], rowsa, sema).wait()
        pltpu.make_async_copy(h.at[idxj], rowsb, semb).wait()

        def _addrow(r, _):
            for j in range(D // 16):
                sl = pl.ds(16 * j, 16)
                rowsa[r, sl] = rowsa[r, sl] + rowsb[r, sl]
            return 0
        lax.fori_loop(0, KE, _addrow, 0)
        pltpu.sync_copy(rowsa, out.at[pl.ds(off, KE)])

    _start(base, idxi0, idxj0, rowsa0, rowsb0, sema0, semb0)

    def _pair(k, _):
        b0 = base + 2 * k * KE
        _start(b0 + KE, idxi1, idxj1, rowsa1, rowsb1, sema1, semb1)
        _finish(b0, idxi0, idxj0, rowsa0, rowsb0, sema0, semb0)
        _start(b0 + 2 * KE, idxi0, idxj0, rowsa0, rowsb0, sema0, semb0)
        _finish(b0 + KE, idxi1, idxj1, rowsa1, rowsb1, sema1, semb1)
        return 0
    lax.fori_loop(0, (nblk - 1) // 2, _pair, 0)
    _finish(base + (nblk - 1) * KE, idxi0, idxj0, rowsa0, rowsb0,
            sema0, semb0)


def _sc_gather_pairsum(h, ni, nj):
    e = ni.shape[0]
    bufset = [
        pltpu.VMEM((KE,), jnp.int32),
        pltpu.VMEM((KE,), jnp.int32),
        pltpu.VMEM((KE, D), jnp.float32),
        pltpu.VMEM((KE, D), jnp.float32),
        pltpu.SemaphoreType.DMA,
        pltpu.SemaphoreType.DMA,
    ]
    f = pl.kernel(
        _sc_gather_body,
        out_type=jax.ShapeDtypeStruct((e, D), jnp.float32),
        mesh=plsc.VectorSubcoreMesh(core_axis_name="c", subcore_axis_name="s",
                                    num_cores=NC, num_subcores=NS),
        scratch_types=bufset + bufset,
    )
    return f(h, ni, nj)


def _edge_out_body(hs_ref, eerp_ref, t_ref):
    t_ref[...] = hs_ref[...] * eerp_ref[...]


def _edge_out(hs, eerp):
    e = hs.shape[0]
    g = e // BEC
    blk = pl.BlockSpec((BEC, D), lambda i: (i, 0))
    return pl.pallas_call(
        _edge_out_body,
        grid=(g,),
        in_specs=[blk, blk],
        out_specs=blk,
        out_shape=jax.ShapeDtypeStruct((e, D), jnp.float32),
    )(hs, eerp)


def kernel(z, p, edge_index, W_ndp, b_ndp, A_nbr, A_na, W1, b1, ln_g, ln_b,
           W2, b2, W_erp, b_erp):
    centers_col = jnp.linspace(_START, 1.0, RBF,
                               dtype=jnp.float32).reshape(RBF, 1)
    zrow = jnp.zeros((KC - RBF - 1, D), jnp.float32)
    w_top = jnp.concatenate([W_ndp, b_ndp.reshape(1, D), zrow], axis=0)
    w_bot = jnp.concatenate([W_erp, b_erp.reshape(1, D), zrow], axis=0)
    zblk = jnp.zeros((KC, D), jnp.float32)
    wcomb = jnp.concatenate(
        [jnp.concatenate([w_top, zblk], axis=1),
         jnp.concatenate([zblk, w_bot], axis=1)], axis=0)

    nj = edge_index[0]
    ni = edge_index[1]
    z2 = z.reshape(-1, 1)
    e = p.shape[0]
    rows = e // 128
    rpad = ((rows + BR - 1) // BR) * BR
    extra = rpad * 128 - e
    px = jnp.pad(p[:, 0], (0, extra)).reshape(rpad, 128)
    py = jnp.pad(p[:, 1], (0, extra)).reshape(rpad, 128)
    pz = jnp.pad(p[:, 2], (0, extra), constant_values=1.0).reshape(rpad, 128)

    sph9, e_ndp, e_erp = _edge_dense(px, py, pz, centers_col, wcomb)
    r_0 = sph9[0, :rows].reshape(e, 1)
    r_ij_1 = sph9[1:4, :rows].reshape(3, e).T
    r_ij_2 = sph9[4:9, :rows].reshape(5, e).T

    anode = _node_embed(z2, A_nbr)
    m_part = _sc_scatter_partials(anode, nj, ni, e_ndp)
    h = _node_mlp(z2, m_part[0], m_part[1], A_na, W1, b1.reshape(1, D),
                  ln_g.reshape(1, D), ln_b.reshape(1, D), W2, b2.reshape(1, D))
    hs = _sc_gather_pairsum(h, ni, nj)
    t_ij = _edge_out(hs, e_erp)
    return (r_0, r_ij_1, r_ij_2, h, t_ij)


# speedup vs baseline: 1.5924x; 1.3297x over previous
import functools
import math

import jax
import jax.numpy as jnp
from jax import lax
from jax.experimental import pallas as pl
from jax.experimental.pallas import tpu as pltpu
from jax.experimental.pallas import tpu_sc as plsc

CUTOFF = 5.0
RBF = 32
D = 128
NA = 100
ALPHA = 5.0 / CUTOFF
_START = math.exp(-CUTOFF)
_DELTA = (1.0 - _START) / (RBF - 1)
GAMMA = (2.0 * _DELTA) ** (-2)
S3 = math.sqrt(3.0)

BR = 32
BE = BR * 128
BEC = 4000
BN = 2000

NC = 2
NS = 16
NW = NC * NS
KE = 80


def _f32(x):
    return x.astype(jnp.float32)


def _cutoff(r):
    return ((jnp.cos((math.pi / CUTOFF) * r) + 1.0) * 0.5) * _f32(r < CUTOFF)


def _rbf(r, centers):
    xe = jnp.exp(-ALPHA * r)
    d = xe - centers
    return jnp.exp(-GAMMA * (d * d)) * _cutoff(r)


KC = 40
KA = 2 * KC + 16
NW3 = 3 * D


def _edge_dense_body(px_ref, py_ref, pz_ref, cen_ref, wc_ref,
                     sph_ref, endp_ref, eerp_ref):
    x = px_ref[...]
    y = py_ref[...]
    z3 = pz_ref[...]
    r = jnp.sqrt(x * x + y * y + z3 * z3)
    inv = 1.0 / r
    ux = x * inv
    uy = y * inv
    uz = z3 * inv
    sph_ref[0] = r
    sph_ref[1] = uy
    sph_ref[2] = uz
    sph_ref[3] = ux
    sph_ref[4] = S3 * ux * uz
    sph_ref[5] = S3 * ux * uy
    sph_ref[6] = uy * uy - 0.5 * (ux * ux + uz * uz)
    sph_ref[7] = S3 * uy * uz
    sph_ref[8] = (S3 / 2.0) * (uz * uz - ux * ux)
    cut = _cutoff(r)
    xe = jnp.exp(-ALPHA * r)
    cenT = cen_ref[...]
    wc = wc_ref[...]
    zero7 = jnp.zeros((KC - RBF - 1, 128), jnp.float32)
    one1 = jnp.ones((1, 128), jnp.float32)
    for i in range(BR):
        sl = slice(i, i + 1)
        xe_i = xe[sl, :]
        cut_i = cut[sl, :]
        dd = cenT - xe_i
        g = jnp.exp(-GAMMA * (dd * dd))
        gn = g * (cut_i * cut_i)
        ge = g * cut_i
        blk = jnp.concatenate(
            [gn, cut_i, zero7, ge, one1, zero7], axis=0)
        out = lax.dot_general(blk, wc, (((0,), (0,)), ((), ())),
                              preferred_element_type=jnp.float32)
        rb = pl.ds(128 * i, 128)
        endp_ref[rb, :] = out[:, :D]
        eerp_ref[rb, :] = out[:, D:]


def _edge_dense(px, py, pz, centers_col, wcomb):
    rows = px.shape[0]
    e = rows * 128
    g = rows // BR
    plane = pl.BlockSpec((BR, 128), lambda i: (i, 0))
    return pl.pallas_call(
        _edge_dense_body,
        grid=(g,),
        in_specs=[
            plane, plane, plane,
            pl.BlockSpec((RBF, 1), lambda i: (0, 0)),
            pl.BlockSpec((2 * KC, 2 * D), lambda i: (0, 0)),
        ],
        out_specs=[
            pl.BlockSpec((9, BR, 128), lambda i: (0, i, 0)),
            pl.BlockSpec((BE, D), lambda i: (i, 0)),
            pl.BlockSpec((BE, D), lambda i: (i, 0)),
        ],
        out_shape=[
            jax.ShapeDtypeStruct((9, rows, 128), jnp.float32),
            jax.ShapeDtypeStruct((e, D), jnp.float32),
            jax.ShapeDtypeStruct((e, D), jnp.float32),
        ],
    )(px, py, pz, centers_col, wcomb)


def _node_embed_body(z_ref, anbr_ref, anode_ref):
    zb = z_ref[...]
    ids = lax.broadcasted_iota(jnp.int32, (1, NA), 1)
    oh = _f32(zb == ids)
    anode_ref[...] = jnp.dot(oh, anbr_ref[...],
                             preferred_element_type=jnp.float32)


def _node_embed(z2, a_nbr):
    n = z2.shape[0]
    g = n // BN
    return pl.pallas_call(
        _node_embed_body,
        grid=(g,),
        in_specs=[
            pl.BlockSpec((BN, 1), lambda i: (i, 0)),
            pl.BlockSpec((NA, D), lambda i: (0, 0)),
        ],
        out_specs=pl.BlockSpec((BN, D), lambda i: (i, 0)),
        out_shape=jax.ShapeDtypeStruct((n, D), jnp.float32),
    )(z2, a_nbr)


def _sc_scatter_body(anode, nj, ni, endp, out,
                     accum, idxj0, idxi0, rows0, erow0, semg0, seme0, sems0,
                     idxj1, idxi1, rows1, erow1, semg1, seme1, sems1, zbuf):
    c = lax.axis_index("c")
    s = lax.axis_index("s")
    wid = s * NC + c
    n_nodes = accum.shape[0]
    epw = nj.shape[0] // NW
    nblk = epw // KE
    chunk = 1000
    ncp = n_nodes // chunk
    zrows = zbuf.shape[0]

    def _zfill(i, _):
        for j in range(D // 16):
            zbuf[i, pl.ds(16 * j, 16)] = jnp.zeros((16,), jnp.float32)
        return 0
    lax.fori_loop(0, zrows, _zfill, 0)

    @pl.when(s < ncp)
    def _():
        def _zcopy(k, _):
            pltpu.sync_copy(zbuf, accum.at[pl.ds(s * chunk + k * zrows, zrows)])
            return 0
        lax.fori_loop(0, chunk // zrows, _zcopy, 0)

    plsc.subcore_barrier()

    base = wid * epw

    sets = ((idxj0, idxi0, rows0, erow0, semg0, seme0, sems0),
            (idxj1, idxi1, rows1, erow1, semg1, seme1, sems1))

    def _start(off, st, wait_scatter):
        idxj, idxi, rows, erow, semg, seme, sems = st
        if wait_scatter:
            pltpu.make_async_copy(rows, accum.at[idxi], sems).wait()
        pltpu.sync_copy(nj.at[pl.ds(off, KE)], idxj)
        pltpu.sync_copy(ni.at[pl.ds(off, KE)], idxi)
        pltpu.async_copy(anode.at[idxj], rows, semg)
        pltpu.async_copy(endp.at[pl.ds(off, KE)], erow, seme)

    def _finish(off, st):
        idxj, idxi, rows, erow, semg, seme, sems = st
        pltpu.make_async_copy(anode.at[idxj], rows, semg).wait()
        pltpu.make_async_copy(endp.at[pl.ds(0, KE)], erow, seme).wait()

        def _mulrow(r, _):
            for j in range(D // 16):
                sl = pl.ds(16 * j, 16)
                rows[r, sl] = rows[r, sl] * erow[r, sl]
            return 0
        lax.fori_loop(0, KE, _mulrow, 0)
        pltpu.async_copy(rows, accum.at[idxi], sems, add=True)

    _start(base, sets[0], False)
    _start(base + KE, sets[1], False)

    def _pair(i, _):
        b = base + 2 * i * KE
        _finish(b, sets[0])
        _start(b + 2 * KE, sets[0], True)
        _finish(b + KE, sets[1])
        _start(b + 3 * KE, sets[1], True)
        return 0
    lax.fori_loop(0, (nblk - 3) // 2, _pair, 0)
    bl = base + (nblk - 3) * KE
    _finish(bl, sets[0])
    _start(bl + 2 * KE, sets[0], True)
    _finish(bl + KE, sets[1])
    _finish(bl + 2 * KE, sets[0])
    pltpu.make_async_copy(rows1, accum.at[idxi1], sems1).wait()
    pltpu.make_async_copy(rows0, accum.at[idxi0], sems0).wait()

    plsc.subcore_barrier()

    @pl.when(s < ncp)
    def _():
        pltpu.sync_copy(accum.at[pl.ds(s * chunk, chunk)],
                        out.at[c, pl.ds(s * chunk, chunk)])


def _sc_scatter_partials(anode, nj, ni, endp):
    n = anode.shape[0]
    bufset = [
        pltpu.VMEM((KE,), jnp.int32),
        pltpu.VMEM((KE,), jnp.int32),
        pltpu.VMEM((KE, D), jnp.float32),
        pltpu.VMEM((KE, D), jnp.float32),
        pltpu.SemaphoreType.DMA,
        pltpu.SemaphoreType.DMA,
        pltpu.SemaphoreType.DMA,
    ]
    f = pl.kernel(
        _sc_scatter_body,
        out_type=jax.ShapeDtypeStruct((NC, n, D), jnp.float32),
        mesh=plsc.VectorSubcoreMesh(core_axis_name="c", subcore_axis_name="s",
                                    num_cores=NC, num_subcores=NS),
        scratch_types=[pltpu.VMEM_SHARED((n, D), jnp.float32)]
        + bufset + bufset + [pltpu.VMEM((40, D), jnp.float32)],
    )
    return f(anode, nj, ni, endp)


def _node_mlp_body(z_ref, m0_ref, m1_ref, ana_ref, w1_ref, b1_ref,
                   g_ref, bb_ref, w2_ref, b2_ref, h_ref):
    zb = z_ref[...]
    ids = lax.broadcasted_iota(jnp.int32, (1, NA), 1)
    oh = _f32(zb == ids)
    a = jnp.dot(oh, ana_ref[...], preferred_element_type=jnp.float32)
    m = m0_ref[...] + m1_ref[...]
    pre = (jnp.dot(a, w1_ref[0:D, :], preferred_element_type=jnp.float32)
           + jnp.dot(m, w1_ref[D:2 * D, :], preferred_element_type=jnp.float32)
           + b1_ref[...])
    mu = jnp.mean(pre, axis=1, keepdims=True)
    cen = pre - mu
    var = jnp.mean(cen * cen, axis=1, keepdims=True)
    xn = cen / jnp.sqrt(var + 1e-5) * g_ref[...] + bb_ref[...]
    h1 = xn / (1.0 + jnp.exp(-xn))
    h_ref[...] = (jnp.dot(h1, w2_ref[...], preferred_element_type=jnp.float32)
                  + b2_ref[...])


def _node_mlp(z2, m0, m1, a_na, w1, b1, ln_g, ln_b, w2, b2):
    n = z2.shape[0]
    g = n // BN
    full = lambda shape: pl.BlockSpec(shape, lambda i: tuple(0 for _ in shape))
    return pl.pallas_call(
        _node_mlp_body,
        grid=(g,),
        in_specs=[
            pl.BlockSpec((BN, 1), lambda i: (i, 0)),
            pl.BlockSpec((BN, D), lambda i: (i, 0)),
            pl.BlockSpec((BN, D), lambda i: (i, 0)),
            full((NA, D)),
            full((2 * D, D)),
            full((1, D)),
            full((1, D)),
            full((1, D)),
            full((D, D)),
            full((1, D)),
        ],
        out_specs=pl.BlockSpec((BN, D), lambda i: (i, 0)),
        out_shape=jax.ShapeDtypeStruct((n, D), jnp.float32),
    )(z2, m0, m1, a_na, w1, b1, ln_g, ln_b, w2, b2)


def _sc_edge_out_body(h, ni, nj, eerp, out,
                      idxi0, idxj0, rowsa0, rowsb0, erow0, sa0, sb0, se0, sw0,
                      idxi1, idxj1, rowsa1, rowsb1, erow1, sa1, sb1, se1, sw1):
    c = lax.axis_index("c")
    s = lax.axis_index("s")
    wid = s * NC + c
    epw = ni.shape[0] // NW
    nblk = epw // KE
    base = wid * epw

    sets = ((idxi0, idxj0, rowsa0, rowsb0, erow0, sa0, sb0, se0, sw0),
            (idxi1, idxj1, rowsa1, rowsb1, erow1, sa1, sb1, se1, sw1))

    def _start(off, st, wait_write):
        idxi, idxj, rowsa, rowsb, erow, sa, sb, se, sw = st
        if wait_write:
            pltpu.make_async_copy(rowsa, out.at[pl.ds(0, KE)], sw).wait()
        pltpu.sync_copy(ni.at[pl.ds(off, KE)], idxi)
        pltpu.sync_copy(nj.at[pl.ds(off, KE)], idxj)
        pltpu.async_copy(h.at[idxi], rowsa, sa)
        pltpu.async_copy(h.at[idxj], rowsb, sb)
        pltpu.async_copy(eerp.at[pl.ds(off, KE)], erow, se)

    def _finish(off, st):
        idxi, idxj, rowsa, rowsb, erow, sa, sb, se, sw = st
        pltpu.make_async_copy(h.at[idxi], rowsa, sa).wait()
        pltpu.make_async_copy(h.at[idxj], rowsb, sb).wait()
        pltpu.make_async_copy(eerp.at[pl.ds(0, KE)], erow, se).wait()

        def _row(r, _):
            for j in range(D // 16):
                sl = pl.ds(16 * j, 16)
                rowsa[r, sl] = (rowsa[r, sl] + rowsb[r, sl]) * erow[r, sl]
            return 0
        lax.fori_loop(0, KE, _row, 0)
        pltpu.async_copy(rowsa, out.at[pl.ds(off, KE)], sw)

    _start(base, sets[0], False)
    _start(base + KE, sets[1], False)

    def _pair(i, _):
        b = base + 2 * i * KE
        _finish(b, sets[0])
        _start(b + 2 * KE, sets[0], True)
        _finish(b + KE, sets[1])
        _start(b + 3 * KE, sets[1], True)
        return 0
    lax.fori_loop(0, (nblk - 3) // 2, _pair, 0)
    bl = base + (nblk - 3) * KE
    _finish(bl, sets[0])
    _start(bl + 2 * KE, sets[0], True)
    _finish(bl + KE, sets[1])
    _finish(bl + 2 * KE, sets[0])
    pltpu.make_async_copy(rowsa1, out.at[pl.ds(0, KE)], sw1).wait()
    pltpu.make_async_copy(rowsa0, out.at[pl.ds(0, KE)], sw0).wait()


def _sc_edge_out(h, ni, nj, eerp):
    e = ni.shape[0]
    bufset = [
        pltpu.VMEM((KE,), jnp.int32),
        pltpu.VMEM((KE,), jnp.int32),
        pltpu.VMEM((KE, D), jnp.float32),
        pltpu.VMEM((KE, D), jnp.float32),
        pltpu.VMEM((KE, D), jnp.float32),
        pltpu.SemaphoreType.DMA,
        pltpu.SemaphoreType.DMA,
        pltpu.SemaphoreType.DMA,
        pltpu.SemaphoreType.DMA,
    ]
    f = pl.kernel(
        _sc_edge_out_body,
        out_type=jax.ShapeDtypeStruct((e, D), jnp.float32),
        mesh=plsc.VectorSubcoreMesh(core_axis_name="c", subcore_axis_name="s",
                                    num_cores=NC, num_subcores=NS),
        scratch_types=bufset + bufset,
    )
    return f(h, ni, nj, eerp)


def kernel(z, p, edge_index, W_ndp, b_ndp, A_nbr, A_na, W1, b1, ln_g, ln_b,
           W2, b2, W_erp, b_erp):
    centers_col = jnp.linspace(_START, 1.0, RBF,
                               dtype=jnp.float32).reshape(RBF, 1)
    zrow = jnp.zeros((KC - RBF - 1, D), jnp.float32)
    w_top = jnp.concatenate([W_ndp, b_ndp.reshape(1, D), zrow], axis=0)
    w_bot = jnp.concatenate([W_erp, b_erp.reshape(1, D), zrow], axis=0)
    zblk = jnp.zeros((KC, D), jnp.float32)
    wcomb = jnp.concatenate(
        [jnp.concatenate([w_top, zblk], axis=1),
         jnp.concatenate([zblk, w_bot], axis=1)], axis=0)

    nj = edge_index[0]
    ni = edge_index[1]
    z2 = z.reshape(-1, 1)
    e = p.shape[0]
    rows = e // 128
    rpad = ((rows + BR - 1) // BR) * BR
    extra = rpad * 128 - e
    px = jnp.pad(p[:, 0], (0, extra)).reshape(rpad, 128)
    py = jnp.pad(p[:, 1], (0, extra)).reshape(rpad, 128)
    pz = jnp.pad(p[:, 2], (0, extra), constant_values=1.0).reshape(rpad, 128)

    sph9, e_ndp, e_erp = _edge_dense(px, py, pz, centers_col, wcomb)
    r_0 = sph9[0, :rows].reshape(e, 1)
    r_ij_1 = sph9[1:4, :rows].reshape(3, e).T
    r_ij_2 = sph9[4:9, :rows].reshape(5, e).T

    anode = _node_embed(z2, A_nbr)
    m_part = _sc_scatter_partials(anode, nj, ni, e_ndp)
    h = _node_mlp(z2, m_part[0], m_part[1], A_na, W1, b1.reshape(1, D),
                  ln_g.reshape(1, D), ln_b.reshape(1, D), W2, b2.reshape(1, D))
    t_ij = _sc_edge_out(h, ni, nj, e_erp)
    return (r_0, r_ij_1, r_ij_2, h, t_ij)


# speedup vs baseline: 1.8418x; 1.1566x over previous
import functools
import math

import jax
import jax.numpy as jnp
from jax import lax
from jax.experimental import pallas as pl
from jax.experimental.pallas import tpu as pltpu
from jax.experimental.pallas import tpu_sc as plsc

CUTOFF = 5.0
RBF = 32
D = 128
NA = 100
ALPHA = 5.0 / CUTOFF
_START = math.exp(-CUTOFF)
_DELTA = (1.0 - _START) / (RBF - 1)
GAMMA = (2.0 * _DELTA) ** (-2)
S3 = math.sqrt(3.0)

BR = 32
BE = BR * 128
BEC = 4000
BN = 2000

NC = 2
NS = 16
NW = NC * NS
KE = 80


def _f32(x):
    return x.astype(jnp.float32)


def _cutoff(r):
    return ((jnp.cos((math.pi / CUTOFF) * r) + 1.0) * 0.5) * _f32(r < CUTOFF)


def _rbf(r, centers):
    xe = jnp.exp(-ALPHA * r)
    d = xe - centers
    return jnp.exp(-GAMMA * (d * d)) * _cutoff(r)


KC = 40
KA = 2 * KC + 16
NW3 = 3 * D


def _edge_dense_body(px_ref, py_ref, pz_ref, cen_ref, wc_ref,
                     sph_ref, endp_ref, eerp_ref):
    x = px_ref[...]
    y = py_ref[...]
    z3 = pz_ref[...]
    r = jnp.sqrt(x * x + y * y + z3 * z3)
    inv = 1.0 / r
    ux = x * inv
    uy = y * inv
    uz = z3 * inv
    sph_ref[0] = r
    sph_ref[1] = uy
    sph_ref[2] = uz
    sph_ref[3] = ux
    sph_ref[4] = S3 * ux * uz
    sph_ref[5] = S3 * ux * uy
    sph_ref[6] = uy * uy - 0.5 * (ux * ux + uz * uz)
    sph_ref[7] = S3 * uy * uz
    sph_ref[8] = (S3 / 2.0) * (uz * uz - ux * ux)
    cut = _cutoff(r)
    xe = jnp.exp(-ALPHA * r)
    cenT = cen_ref[...]
    wc = wc_ref[...]
    zero7 = jnp.zeros((KC - RBF - 1, 128), jnp.float32)
    one1 = jnp.ones((1, 128), jnp.float32)
    for i in range(BR):
        sl = slice(i, i + 1)
        xe_i = xe[sl, :]
        cut_i = cut[sl, :]
        dd = cenT - xe_i
        g = jnp.exp(-GAMMA * (dd * dd))
        gn = g * (cut_i * cut_i)
        ge = g * cut_i
        blk = jnp.concatenate(
            [gn, cut_i, zero7, ge, one1, zero7], axis=0)
        out = lax.dot_general(blk, wc, (((0,), (0,)), ((), ())),
                              preferred_element_type=jnp.float32)
        rb = pl.ds(128 * i, 128)
        endp_ref[rb, :] = out[:, :D]
        eerp_ref[rb, :] = out[:, D:]


def _edge_dense(px, py, pz, centers_col, wcomb):
    rows = px.shape[0]
    e = rows * 128
    g = rows // BR
    plane = pl.BlockSpec((BR, 128), lambda i: (i, 0))
    return pl.pallas_call(
        _edge_dense_body,
        grid=(g,),
        in_specs=[
            plane, plane, plane,
            pl.BlockSpec((RBF, 1), lambda i: (0, 0)),
            pl.BlockSpec((2 * KC, 2 * D), lambda i: (0, 0)),
        ],
        out_specs=[
            pl.BlockSpec((9, BR, 128), lambda i: (0, i, 0)),
            pl.BlockSpec((BE, D), lambda i: (i, 0)),
            pl.BlockSpec((BE, D), lambda i: (i, 0)),
        ],
        out_shape=[
            jax.ShapeDtypeStruct((9, rows, 128), jnp.float32),
            jax.ShapeDtypeStruct((e, D), jnp.float32),
            jax.ShapeDtypeStruct((e, D), jnp.float32),
        ],
    )(px, py, pz, centers_col, wcomb)


def _node_embed_body(z_ref, anbr_ref, anode_ref):
    zb = z_ref[...]
    ids = lax.broadcasted_iota(jnp.int32, (1, NA), 1)
    oh = _f32(zb == ids)
    anode_ref[...] = jnp.dot(oh, anbr_ref[...],
                             preferred_element_type=jnp.float32)


def _node_embed(z2, a_nbr):
    n = z2.shape[0]
    g = n // BN
    return pl.pallas_call(
        _node_embed_body,
        grid=(g,),
        in_specs=[
            pl.BlockSpec((BN, 1), lambda i: (i, 0)),
            pl.BlockSpec((NA, D), lambda i: (0, 0)),
        ],
        out_specs=pl.BlockSpec((BN, D), lambda i: (i, 0)),
        out_shape=jax.ShapeDtypeStruct((n, D), jnp.float32),
    )(z2, a_nbr)


def _sc_scatter_body(anode, nj, ni, endp, out,
                     accum, idxj0, idxi0, rows0, erow0, semg0, seme0, sems0,
                     idxj1, idxi1, rows1, erow1, semg1, seme1, sems1, zbuf):
    c = lax.axis_index("c")
    s = lax.axis_index("s")
    wid = s * NC + c
    n_nodes = accum.shape[0]
    epw = nj.shape[0] // NW
    nblk = epw // KE
    chunk = 1000
    ncp = n_nodes // chunk
    zrows = zbuf.shape[0]

    def _zfill(i, _):
        for j in range(D // 16):
            zbuf[i, pl.ds(16 * j, 16)] = jnp.zeros((16,), jnp.float32)
        return 0
    lax.fori_loop(0, zrows, _zfill, 0)

    @pl.when(s < ncp)
    def _():
        def _zcopy(k, _):
            pltpu.sync_copy(zbuf, accum.at[pl.ds(s * chunk + k * zrows, zrows)])
            return 0
        lax.fori_loop(0, chunk // zrows, _zcopy, 0)

    plsc.subcore_barrier()

    base = wid * epw

    sets = ((idxj0, idxi0, rows0, erow0, semg0, seme0, sems0),
            (idxj1, idxi1, rows1, erow1, semg1, seme1, sems1))

    def _start(off, st, wait_scatter):
        idxj, idxi, rows, erow, semg, seme, sems = st
        if wait_scatter:
            pltpu.make_async_copy(rows, accum.at[idxi], sems).wait()
        pltpu.sync_copy(nj.at[pl.ds(off, KE)], idxj)
        pltpu.sync_copy(ni.at[pl.ds(off, KE)], idxi)
        pltpu.async_copy(anode.at[idxj], rows, semg)
        pltpu.async_copy(endp.at[pl.ds(off, KE)], erow, seme)

    def _finish(off, st):
        idxj, idxi, rows, erow, semg, seme, sems = st
        pltpu.make_async_copy(anode.at[idxj], rows, semg).wait()
        pltpu.make_async_copy(endp.at[pl.ds(0, KE)], erow, seme).wait()

        @functools.partial(plsc.parallel_loop, 0, KE, unroll=2)
        def _mulrow(r):
            for j in range(D // 16):
                sl = pl.ds(16 * j, 16)
                rows[r, sl] = rows[r, sl] * erow[r, sl]
        pltpu.async_copy(rows, accum.at[idxi], sems, add=True)

    _start(base, sets[0], False)
    _start(base + KE, sets[1], False)

    def _pair(i, _):
        b = base + 2 * i * KE
        _finish(b, sets[0])
        _start(b + 2 * KE, sets[0], True)
        _finish(b + KE, sets[1])
        _start(b + 3 * KE, sets[1], True)
        return 0
    lax.fori_loop(0, (nblk - 3) // 2, _pair, 0)
    bl = base + (nblk - 3) * KE
    _finish(bl, sets[0])
    _start(bl + 2 * KE, sets[0], True)
    _finish(bl + KE, sets[1])
    _finish(bl + 2 * KE, sets[0])
    pltpu.make_async_copy(rows1, accum.at[idxi1], sems1).wait()
    pltpu.make_async_copy(rows0, accum.at[idxi0], sems0).wait()

    plsc.subcore_barrier()

    @pl.when(s < ncp)
    def _():
        pltpu.sync_copy(accum.at[pl.ds(s * chunk, chunk)],
                        out.at[c, pl.ds(s * chunk, chunk)])


def _sc_scatter_partials(anode, nj, ni, endp):
    n = anode.shape[0]
    bufset = [
        pltpu.VMEM((KE,), jnp.int32),
        pltpu.VMEM((KE,), jnp.int32),
        pltpu.VMEM((KE, D), jnp.float32),
        pltpu.VMEM((KE, D), jnp.float32),
        pltpu.SemaphoreType.DMA,
        pltpu.SemaphoreType.DMA,
        pltpu.SemaphoreType.DMA,
    ]
    f = pl.kernel(
        _sc_scatter_body,
        out_type=jax.ShapeDtypeStruct((NC, n, D), jnp.float32),
        mesh=plsc.VectorSubcoreMesh(core_axis_name="c", subcore_axis_name="s",
                                    num_cores=NC, num_subcores=NS),
        scratch_types=[pltpu.VMEM_SHARED((n, D), jnp.float32)]
        + bufset + bufset + [pltpu.VMEM((40, D), jnp.float32)],
    )
    return f(anode, nj, ni, endp)


def _node_mlp_body(z_ref, m0_ref, m1_ref, ana_ref, w1_ref, b1_ref,
                   g_ref, bb_ref, w2_ref, b2_ref, h_ref):
    zb = z_ref[...]
    ids = lax.broadcasted_iota(jnp.int32, (1, NA), 1)
    oh = _f32(zb == ids)
    a = jnp.dot(oh, ana_ref[...], preferred_element_type=jnp.float32)
    m = m0_ref[...] + m1_ref[...]
    pre = (jnp.dot(a, w1_ref[0:D, :], preferred_element_type=jnp.float32)
           + jnp.dot(m, w1_ref[D:2 * D, :], preferred_element_type=jnp.float32)
           + b1_ref[...])
    mu = jnp.mean(pre, axis=1, keepdims=True)
    cen = pre - mu
    var = jnp.mean(cen * cen, axis=1, keepdims=True)
    xn = cen / jnp.sqrt(var + 1e-5) * g_ref[...] + bb_ref[...]
    h1 = xn / (1.0 + jnp.exp(-xn))
    h_ref[...] = (jnp.dot(h1, w2_ref[...], preferred_element_type=jnp.float32)
                  + b2_ref[...])


def _node_mlp(z2, m0, m1, a_na, w1, b1, ln_g, ln_b, w2, b2):
    n = z2.shape[0]
    g = n // BN
    full = lambda shape: pl.BlockSpec(shape, lambda i: tuple(0 for _ in shape))
    return pl.pallas_call(
        _node_mlp_body,
        grid=(g,),
        in_specs=[
            pl.BlockSpec((BN, 1), lambda i: (i, 0)),
            pl.BlockSpec((BN, D), lambda i: (i, 0)),
            pl.BlockSpec((BN, D), lambda i: (i, 0)),
            full((NA, D)),
            full((2 * D, D)),
            full((1, D)),
            full((1, D)),
            full((1, D)),
            full((D, D)),
            full((1, D)),
        ],
        out_specs=pl.BlockSpec((BN, D), lambda i: (i, 0)),
        out_shape=jax.ShapeDtypeStruct((n, D), jnp.float32),
    )(z2, m0, m1, a_na, w1, b1, ln_g, ln_b, w2, b2)


def _sc_edge_out_body(h, ni, nj, eerp, out,
                      idxi0, idxj0, rowsa0, rowsb0, erow0, sa0, sb0, se0, sw0,
                      idxi1, idxj1, rowsa1, rowsb1, erow1, sa1, sb1, se1, sw1):
    c = lax.axis_index("c")
    s = lax.axis_index("s")
    wid = s * NC + c
    epw = ni.shape[0] // NW
    nblk = epw // KE
    base = wid * epw

    sets = ((idxi0, idxj0, rowsa0, rowsb0, erow0, sa0, sb0, se0, sw0),
            (idxi1, idxj1, rowsa1, rowsb1, erow1, sa1, sb1, se1, sw1))

    def _start(off, st, wait_write):
        idxi, idxj, rowsa, rowsb, erow, sa, sb, se, sw = st
        if wait_write:
            pltpu.make_async_copy(rowsa, out.at[pl.ds(0, KE)], sw).wait()
        pltpu.sync_copy(ni.at[pl.ds(off, KE)], idxi)
        pltpu.sync_copy(nj.at[pl.ds(off, KE)], idxj)
        pltpu.async_copy(h.at[idxi], rowsa, sa)
        pltpu.async_copy(h.at[idxj], rowsb, sb)
        pltpu.async_copy(eerp.at[pl.ds(off, KE)], erow, se)

    def _finish(off, st):
        idxi, idxj, rowsa, rowsb, erow, sa, sb, se, sw = st
        pltpu.make_async_copy(h.at[idxi], rowsa, sa).wait()
        pltpu.make_async_copy(h.at[idxj], rowsb, sb).wait()
        pltpu.make_async_copy(eerp.at[pl.ds(0, KE)], erow, se).wait()

        @functools.partial(plsc.parallel_loop, 0, KE, unroll=2)
        def _row(r):
            for j in range(D // 16):
                sl = pl.ds(16 * j, 16)
                rowsa[r, sl] = (rowsa[r, sl] + rowsb[r, sl]) * erow[r, sl]
        pltpu.async_copy(rowsa, out.at[pl.ds(off, KE)], sw)

    _start(base, sets[0], False)
    _start(base + KE, sets[1], False)

    def _pair(i, _):
        b = base + 2 * i * KE
        _finish(b, sets[0])
        _start(b + 2 * KE, sets[0], True)
        _finish(b + KE, sets[1])
        _start(b + 3 * KE, sets[1], True)
        return 0
    lax.fori_loop(0, (nblk - 3) // 2, _pair, 0)
    bl = base + (nblk - 3) * KE
    _finish(bl, sets[0])
    _start(bl + 2 * KE, sets[0], True)
    _finish(bl + KE, sets[1])
    _finish(bl + 2 * KE, sets[0])
    pltpu.make_async_copy(rowsa1, out.at[pl.ds(0, KE)], sw1).wait()
    pltpu.make_async_copy(rowsa0, out.at[pl.ds(0, KE)], sw0).wait()


def _sc_edge_out(h, ni, nj, eerp):
    e = ni.shape[0]
    bufset = [
        pltpu.VMEM((KE,), jnp.int32),
        pltpu.VMEM((KE,), jnp.int32),
        pltpu.VMEM((KE, D), jnp.float32),
        pltpu.VMEM((KE, D), jnp.float32),
        pltpu.VMEM((KE, D), jnp.float32),
        pltpu.SemaphoreType.DMA,
        pltpu.SemaphoreType.DMA,
        pltpu.SemaphoreType.DMA,
        pltpu.SemaphoreType.DMA,
    ]
    f = pl.kernel(
        _sc_edge_out_body,
        out_type=jax.ShapeDtypeStruct((e, D), jnp.float32),
        mesh=plsc.VectorSubcoreMesh(core_axis_name="c", subcore_axis_name="s",
                                    num_cores=NC, num_subcores=NS),
        scratch_types=bufset + bufset,
    )
    return f(h, ni, nj, eerp)


def kernel(z, p, edge_index, W_ndp, b_ndp, A_nbr, A_na, W1, b1, ln_g, ln_b,
           W2, b2, W_erp, b_erp):
    centers_col = jnp.linspace(_START, 1.0, RBF,
                               dtype=jnp.float32).reshape(RBF, 1)
    zrow = jnp.zeros((KC - RBF - 1, D), jnp.float32)
    w_top = jnp.concatenate([W_ndp, b_ndp.reshape(1, D), zrow], axis=0)
    w_bot = jnp.concatenate([W_erp, b_erp.reshape(1, D), zrow], axis=0)
    zblk = jnp.zeros((KC, D), jnp.float32)
    wcomb = jnp.concatenate(
        [jnp.concatenate([w_top, zblk], axis=1),
         jnp.concatenate([zblk, w_bot], axis=1)], axis=0)

    nj = edge_index[0]
    ni = edge_index[1]
    z2 = z.reshape(-1, 1)
    e = p.shape[0]
    rows = e // 128
    rpad = ((rows + BR - 1) // BR) * BR
    extra = rpad * 128 - e
    px = jnp.pad(p[:, 0], (0, extra)).reshape(rpad, 128)
    py = jnp.pad(p[:, 1], (0, extra)).reshape(rpad, 128)
    pz = jnp.pad(p[:, 2], (0, extra), constant_values=1.0).reshape(rpad, 128)

    sph9, e_ndp, e_erp = _edge_dense(px, py, pz, centers_col, wcomb)
    r_0 = sph9[0, :rows].reshape(e, 1)
    r_ij_1 = sph9[1:4, :rows].reshape(3, e).T
    r_ij_2 = sph9[4:9, :rows].reshape(5, e).T

    anode = _node_embed(z2, A_nbr)
    m_part = _sc_scatter_partials(anode, nj, ni, e_ndp)
    h = _node_mlp(z2, m_part[0], m_part[1], A_na, W1, b1.reshape(1, D),
                  ln_g.reshape(1, D), ln_b.reshape(1, D), W2, b2.reshape(1, D))
    t_ij = _sc_edge_out(h, ni, nj, e_erp)
    return (r_0, r_ij_1, r_ij_2, h, t_ij)
